# baseline (speedup 1.0000x reference)
import jax
import jax.numpy as jnp
from jax.experimental import pallas as pl
from jax.experimental.pallas import tpu as pltpu

_N = 256
_TI = 8


def _gat_layer(h, Wla, Wra, Wr, b_row, bd):
    hlp = jnp.dot(h, Wla, preferred_element_type=jnp.float32)
    hrp = jnp.dot(h, Wra, preferred_element_type=jnp.float32)
    hr = jnp.dot(h, Wr, preferred_element_type=jnp.float32)
    sl = jnp.sum(hlp, axis=1, keepdims=True)
    hrpT = hrp.T
    srT = jnp.sum(hrpT, axis=0, keepdims=True)
    hlp16 = hlp.astype(jnp.bfloat16)
    hrpT16 = hrpT.astype(jnp.bfloat16)
    base = 0.6 * (sl + srT)
    nums, sums = [], []
    for c0 in range(0, _N, 64):
        rows = []
        for i0 in range(c0, c0 + 64, _TI):
            u = hlp16[i0:i0 + _TI, :, None] + hrpT16[None, :, :]
            t = jnp.abs(u)
            rows.append(jnp.dot(bd, t.reshape(_TI * 64, _N),
                                preferred_element_type=jnp.float32))
        sc = jnp.concatenate(rows, axis=0) + base[c0:c0 + 64, :]
        m = jnp.max(sc, axis=-1, keepdims=True)
        p = jnp.exp(sc - m)
        nums.append(p)
        sums.append(jnp.sum(p, axis=-1, keepdims=True))
    num = jnp.concatenate(nums, axis=0)
    inv = 1.0 / jnp.concatenate(sums, axis=0)
    out = jnp.dot(num, hr, preferred_element_type=jnp.float32) * inv + b_row
    return out


def _fused_kernel(x_ref, w1la_ref, w1ra_ref, bd1_ref, w1r_ref, b1_ref,
                  w2la_ref, w2ra_ref, bd2_ref, w2r_ref, b2_ref, out_ref):
    x = x_ref[0]
    h = jnp.tanh(_gat_layer(x, w1la_ref[0], w1ra_ref[0],
                            w1r_ref[0], b1_ref[0], bd1_ref[0]))
    h = jnp.tanh(_gat_layer(h, w2la_ref[0], w2ra_ref[0],
                            w2r_ref[0], b2_ref[0], bd2_ref[0]))
    out_ref[0, 0] = jnp.mean(h, axis=0, keepdims=True)


def kernel(features, p1_Wl, p1_Wr, p1_a, p1_b, p2_Wl, p2_Wr, p2_a, p2_b,
           v1_Wl, v1_Wr, v1_a, v1_b, v2_Wl, v2_Wr, v2_a, v2_b):
    B, N, F = features.shape
    D = p1_Wl.shape[1]

    eye = (jnp.arange(_TI)[:, None] ==
           (jnp.arange(_TI * D) // D)[None, :]).astype(jnp.float32)

    def prep(Wl, Wr, a, b):
        bds = (eye * jnp.tile(0.4 * jnp.sign(a), (_TI,))[None, :]
               ).astype(jnp.bfloat16)
        return (Wl * a[None, :], Wr * a[None, :], bds, Wr, b[None, :])

    l1 = [prep(p1_Wl, p1_Wr, p1_a, p1_b), prep(v1_Wl, v1_Wr, v1_a, v1_b)]
    l2 = [prep(p2_Wl, p2_Wr, p2_a, p2_b), prep(v2_Wl, v2_Wr, v2_a, v2_b)]
    w1la, w1ra, s1, w1r, b1 = (jnp.stack([t[k] for t in l1]) for k in range(5))
    w2la, w2ra, s2, w2r, b2 = (jnp.stack([t[k] for t in l2]) for k in range(5))

    out = pl.pallas_call(
        _fused_kernel,
        grid=(2, B),
        in_specs=[
            pl.BlockSpec((1, N, F), lambda n, b: (b, 0, 0)),
            pl.BlockSpec((1, F, D), lambda n, b: (n, 0, 0)),
            pl.BlockSpec((1, F, D), lambda n, b: (n, 0, 0)),
            pl.BlockSpec((1, _TI, _TI * D), lambda n, b: (n, 0, 0)),
            pl.BlockSpec((1, F, D), lambda n, b: (n, 0, 0)),
            pl.BlockSpec((1, 1, D), lambda n, b: (n, 0, 0)),
            pl.BlockSpec((1, D, D), lambda n, b: (n, 0, 0)),
            pl.BlockSpec((1, D, D), lambda n, b: (n, 0, 0)),
            pl.BlockSpec((1, _TI, _TI * D), lambda n, b: (n, 0, 0)),
            pl.BlockSpec((1, D, D), lambda n, b: (n, 0, 0)),
            pl.BlockSpec((1, 1, D), lambda n, b: (n, 0, 0)),
        ],
        out_specs=pl.BlockSpec((1, 1, 1, D), lambda n, b: (n, b, 0, 0)),
        out_shape=jax.ShapeDtypeStruct((2, B, 1, D), jnp.float32),
        compiler_params=pltpu.CompilerParams(
            dimension_semantics=("parallel", "parallel")),
    )(features, w1la, w1ra, s1, w1r, b1, w2la, w2ra, s2, w2r, b2)
    out = out.reshape(2, B, D)
    return out[0], out[1]


# speedup vs baseline: 1.0025x; 1.0025x over previous
import jax
import jax.numpy as jnp
from jax.experimental import pallas as pl
from jax.experimental.pallas import tpu as pltpu

_N = 256
_TI = 8


def _gat_layer(h, Wla, Wra, Wr, b_row, bd):
    hlp = jnp.dot(h, Wla, preferred_element_type=jnp.float32)
    hrp = jnp.dot(h, Wra, preferred_element_type=jnp.float32)
    hr = jnp.dot(h, Wr, preferred_element_type=jnp.float32)
    sl = jnp.sum(hlp, axis=1, keepdims=True)
    hrpT = hrp.T
    srT = jnp.sum(hrpT, axis=0, keepdims=True)
    hlp16 = hlp.astype(jnp.bfloat16)
    hrpT16 = hrpT.astype(jnp.bfloat16)
    base = 0.6 * (sl + srT)
    nums, sums = [], []
    for c0 in range(0, _N, 64):
        rows = []
        for i0 in range(c0, c0 + 64, _TI):
            u = hlp16[i0:i0 + _TI, :, None] + hrpT16[None, :, :]
            t = jnp.abs(u)
            rows.append(jnp.dot(bd, t.reshape(_TI * 64, _N),
                                preferred_element_type=jnp.float32))
        sc = jnp.concatenate(rows, axis=0) + base[c0:c0 + 64, :]
        m = jnp.max(sc, axis=-1, keepdims=True)
        p = jnp.exp(sc - m)
        nums.append(p)
        sums.append(jnp.sum(p, axis=-1, keepdims=True))
    num = jnp.concatenate(nums, axis=0)
    inv = 1.0 / jnp.concatenate(sums, axis=0)
    out = jnp.dot(num, hr, preferred_element_type=jnp.float32) * inv + b_row
    return out


def _fused_kernel(x_ref, w1la_ref, w1ra_ref, bd1_ref, w1r_ref, b1_ref,
                  w2la_ref, w2ra_ref, bd2_ref, w2r_ref, b2_ref, out_ref):
    x = x_ref[0]
    h = jnp.tanh(_gat_layer(x, w1la_ref[0], w1ra_ref[0],
                            w1r_ref[0], b1_ref[0], bd1_ref[0]))
    h = jnp.tanh(_gat_layer(h, w2la_ref[0], w2ra_ref[0],
                            w2r_ref[0], b2_ref[0], bd2_ref[0]))
    out_ref[0, 0] = jnp.mean(h, axis=0, keepdims=True)


def kernel(features, p1_Wl, p1_Wr, p1_a, p1_b, p2_Wl, p2_Wr, p2_a, p2_b,
           v1_Wl, v1_Wr, v1_a, v1_b, v2_Wl, v2_Wr, v2_a, v2_b):
    B, N, F = features.shape
    D = p1_Wl.shape[1]

    eye = (jnp.arange(_TI)[:, None] ==
           (jnp.arange(_TI * D) // D)[None, :]).astype(jnp.float32)

    def prep(Wl, Wr, a, b):
        bds = (eye * jnp.tile(0.4 * jnp.sign(a), (_TI,))[None, :]
               ).astype(jnp.bfloat16)
        return (Wl * a[None, :], Wr * a[None, :], bds, Wr, b[None, :])

    l1 = [prep(p1_Wl, p1_Wr, p1_a, p1_b), prep(v1_Wl, v1_Wr, v1_a, v1_b)]
    l2 = [prep(p2_Wl, p2_Wr, p2_a, p2_b), prep(v2_Wl, v2_Wr, v2_a, v2_b)]
    w1la, w1ra, s1, w1r, b1 = (jnp.stack([t[k] for t in l1]) for k in range(5))
    w2la, w2ra, s2, w2r, b2 = (jnp.stack([t[k] for t in l2]) for k in range(5))

    out = pl.pallas_call(
        _fused_kernel,
        grid=(2, B),
        in_specs=[
            pl.BlockSpec((1, N, F), lambda n, b: (b, 0, 0)),
            pl.BlockSpec((1, F, D), lambda n, b: (n, 0, 0)),
            pl.BlockSpec((1, F, D), lambda n, b: (n, 0, 0)),
            pl.BlockSpec((1, _TI, _TI * D), lambda n, b: (n, 0, 0)),
            pl.BlockSpec((1, F, D), lambda n, b: (n, 0, 0)),
            pl.BlockSpec((1, 1, D), lambda n, b: (n, 0, 0)),
            pl.BlockSpec((1, D, D), lambda n, b: (n, 0, 0)),
            pl.BlockSpec((1, D, D), lambda n, b: (n, 0, 0)),
            pl.BlockSpec((1, _TI, _TI * D), lambda n, b: (n, 0, 0)),
            pl.BlockSpec((1, D, D), lambda n, b: (n, 0, 0)),
            pl.BlockSpec((1, 1, D), lambda n, b: (n, 0, 0)),
        ],
        out_specs=pl.BlockSpec((1, 1, 1, D), lambda n, b: (n, b, 0, 0)),
        out_shape=jax.ShapeDtypeStruct((2, B, 1, D), jnp.float32),
        compiler_params=pltpu.CompilerParams(
            dimension_semantics=("arbitrary", "arbitrary")),
    )(features, w1la, w1ra, s1, w1r, b1, w2la, w2ra, s2, w2r, b2)
    out = out.reshape(2, B, D)
    return out[0], out[1]
